# baseline (speedup 1.0000x reference)
import jax
import jax.numpy as jnp
from jax import lax
from jax.experimental import pallas as pl
from jax.experimental.pallas import tpu as pltpu
from jax.experimental.pallas import tpu_sc as plsc

_NC = 2
_NS = 16
_L = 16
_NW = _NC * _NS

_N = 16384
_F = 64
_C = _F // _L
_ROWS = _N // _NW
_GROUPS = _ROWS // _L


def _body(x_hbm, p_hbm, out_hbm, x_v, p_v, o_v):
    wid = lax.axis_index("s") * _NC + lax.axis_index("c")
    base = wid * _ROWS * _F
    pltpu.sync_copy(x_hbm.at[pl.ds(base, _ROWS * _F)], x_v)
    pltpu.sync_copy(p_hbm, p_v)

    lanes = lax.iota(jnp.int32, _L)
    ks = [p_v[0, pl.ds(c * _L, _L)] for c in range(_C)]
    ms = [p_v[1, pl.ds(c * _L, _L)] for c in range(_C)]

    def group(g, _):
        g0 = g * (_L * _F)
        e = jnp.zeros((_L,), jnp.float32)
        for r in range(_L):
            pr = None
            for c in range(_C):
                xv = x_v[pl.ds(g0 + r * _F + c * _L, _L)]
                d = xv - ms[c]
                t = ks[c] * d * d
                pr = t if pr is None else pr + t
            e = jnp.where(lanes == r, jnp.sum(pr), e)
        o_v[pl.ds(g * _L, _L)] = e
        return _

    lax.fori_loop(0, _GROUPS, group, None)
    pltpu.sync_copy(o_v, out_hbm.at[pl.ds(wid * _ROWS, _ROWS)])


@jax.jit
def _sc_call(xflat, p):
    mesh = plsc.VectorSubcoreMesh(
        core_axis_name="c", subcore_axis_name="s",
        num_cores=_NC, num_subcores=_NS,
    )
    f = pl.kernel(
        _body,
        out_type=jax.ShapeDtypeStruct((_N,), jnp.float32),
        mesh=mesh,
        scratch_types=[
            pltpu.VMEM((_ROWS * _F,), jnp.float32),
            pltpu.VMEM((2, _F), jnp.float32),
            pltpu.VMEM((_ROWS,), jnp.float32),
        ],
        compiler_params=pltpu.CompilerParams(needs_layout_passes=False),
    )
    return f(xflat, p)


def kernel(in_feat, harmonic_parameters):
    n, f = in_feat.shape
    p = jnp.stack([
        0.5 * harmonic_parameters[0, :],
        harmonic_parameters[1, :],
    ])
    out = _sc_call(in_feat.reshape(-1), p)
    return out.reshape(n, 1)


# speedup vs baseline: 1.4385x; 1.4385x over previous
import jax
import jax.numpy as jnp
from jax.experimental import pallas as pl
from jax.experimental.pallas import tpu as pltpu


_NCHUNK = 8


def _body(x_hbm, hp_ref, sel_ref, out_ref, x_vmem, sems):
    nv, fv = x_vmem.shape
    rows = nv // _NCHUNK

    def copy(c):
        return pltpu.make_async_copy(
            x_hbm.at[pl.ds(c * rows, rows), :],
            x_vmem.at[pl.ds(c * rows, rows), :],
            sems.at[c],
        )

    for c in range(_NCHUNK):
        copy(c).start()

    k = hp_ref[0, :]
    m = hp_ref[1, :]
    km = k * m
    a = 0.5 * k
    const = 0.25 * jnp.sum(km * m)
    for c in range(_NCHUNK):
        copy(c).wait()
        x = x_vmem[pl.ds(c * rows, rows), :]
        t = x * (a[None, :] * x - km[None, :])
        tt = t.T
        e2 = jax.lax.dot_general(
            sel_ref[...], tt, (((1,), (0,)), ((), ())),
            preferred_element_type=jnp.float32,
        )
        out_ref[:, pl.ds(c * rows, rows)] = e2 + const


def kernel(in_feat, harmonic_parameters):
    n, f = in_feat.shape
    n2, f2 = n // 2, f * 2
    xr = in_feat.reshape(n2, f2)
    hp2 = jnp.tile(harmonic_parameters, (1, 2))
    half = (jax.lax.iota(jnp.int32, f2) >= f).astype(jnp.float32)
    sel = jnp.stack([1.0 - half, half], axis=0)
    out = pl.pallas_call(
        _body,
        in_specs=[
            pl.BlockSpec(memory_space=pltpu.MemorySpace.HBM),
            pl.BlockSpec((2, f2), lambda: (0, 0)),
            pl.BlockSpec((2, f2), lambda: (0, 0)),
        ],
        out_specs=pl.BlockSpec((2, n2), lambda: (0, 0)),
        out_shape=jax.ShapeDtypeStruct((2, n2), jnp.float32),
        scratch_shapes=[
            pltpu.VMEM((n2, f2), jnp.float32),
            pltpu.SemaphoreType.DMA((_NCHUNK,)),
        ],
        compiler_params=pltpu.CompilerParams(
            allow_input_fusion=[True, False, False],
        ),
        grid=(),
    )(xr, hp2, sel)
    return out.T.reshape(n, 1)
